# baseline (speedup 1.0000x reference)
import functools

import jax
import jax.numpy as jnp
from jax import lax
from jax.experimental import pallas as pl
from jax.experimental.pallas import tpu as pltpu
from jax.experimental.pallas import tpu_sc as plsc

N_VOCAB = 1000000
N_EMBED = 64
BATCH = 16384

_info = plsc.get_sparse_core_info()
_NC, _NS, _L = _info.num_cores, _info.num_subcores, _info.num_lanes
_NW = _NC * _NS
_BPW = BATCH // _NW
_CHUNK = 128
_NCHUNK = _BPW // _CHUNK

_mesh = plsc.VectorSubcoreMesh(core_axis_name="c", subcore_axis_name="s")


@functools.partial(
    pl.kernel,
    mesh=_mesh,
    out_type=jax.ShapeDtypeStruct((BATCH, N_EMBED), jnp.float32),
    scratch_types=[
        pltpu.VMEM((_BPW,), jnp.int32),
        pltpu.VMEM((_BPW,), jnp.int32),
        pltpu.VMEM((_CHUNK, 2 * N_EMBED), jnp.float32),
        pltpu.VMEM((_CHUNK, N_EMBED), jnp.float32),
        pltpu.SemaphoreType.DMA,
    ],
    compiler_params=pltpu.CompilerParams(
        use_tc_tiling_on_sc=False, needs_layout_passes=False
    ),
)
def _gather_kernel(idx_hbm, tbl_hbm, out_hbm,
                   idx_v, pair_v, pairs_v, rows_v, sem):
    wid = lax.axis_index("s") * _NC + lax.axis_index("c")
    base = wid * _BPW
    pltpu.sync_copy(idx_hbm.at[pl.ds(base, _BPW)], idx_v)
    for t in range(_BPW // _L):
        v = idx_v[pl.ds(t * _L, _L)]
        pair_v[pl.ds(t * _L, _L)] = lax.shift_right_logical(v, 1)

    iota = lax.iota(jnp.int32, _L)
    for j in range(_NCHUNK):
        pltpu.async_copy(
            tbl_hbm.at[pair_v.at[pl.ds(j * _CHUNK, _CHUNK)]],
            pairs_v, sem,
        ).wait()
        for g in range(_CHUNK // _L):
            sub = lax.bitwise_and(idx_v[pl.ds(j * _CHUNK + g * _L, _L)], 1)
            pos = iota + g * _L

            off = sub * N_EMBED

            def body(q, _, off=off, pos=pos):
                for dd in range(4):
                    d = q * 4 + dd
                    dvec = jnp.full((_L,), d, jnp.int32)
                    vals = plsc.load_gather(pairs_v, [pos, off + dvec])
                    plsc.store_scatter(rows_v, [pos, dvec], vals)
                return 0

            lax.fori_loop(0, N_EMBED // 4, body, 0)
        pltpu.sync_copy(rows_v, out_hbm.at[pl.ds(base + j * _CHUNK, _CHUNK)])


def kernel(input_words, in_embed):
    idx = input_words.astype(jnp.int32)
    tbl = in_embed.reshape(N_VOCAB // 2, 2 * N_EMBED)
    return _gather_kernel(idx, tbl)


# speedup vs baseline: 1.7800x; 1.7800x over previous
import functools

import jax
import jax.numpy as jnp
from jax import lax
from jax.experimental import pallas as pl
from jax.experimental.pallas import tpu as pltpu
from jax.experimental.pallas import tpu_sc as plsc

N_VOCAB = 1000000
N_EMBED = 64
BATCH = 16384

_info = plsc.get_sparse_core_info()
_NC, _NS, _L = _info.num_cores, _info.num_subcores, _info.num_lanes
_NW = _NC * _NS
_BPW = BATCH // _NW
_CHUNK = 64
_NCHUNK = _BPW // (2 * _CHUNK)

_mesh = plsc.VectorSubcoreMesh(core_axis_name="c", subcore_axis_name="s")


@functools.partial(
    pl.kernel,
    mesh=_mesh,
    out_type=jax.ShapeDtypeStruct((BATCH, N_EMBED), jnp.float32),
    scratch_types=[
        pltpu.VMEM((_BPW,), jnp.int32),
        pltpu.VMEM((2, _CHUNK, N_EMBED), jnp.float32),
        pltpu.VMEM_SHARED((_NS, 2, _CHUNK, N_EMBED), jnp.float32),
        [pltpu.SemaphoreType.DMA] * 2,
        [pltpu.SemaphoreType.DMA] * 2,
        pltpu.SemaphoreType.DMA,
    ],
)
def _gather_kernel(idx_hbm, tbl_hbm, out_hbm,
                   idx_v, rows_v, rows_sh, vsems, ssems, osem):
    cid = lax.axis_index("c")
    sid = lax.axis_index("s")
    wid = sid * _NC + cid
    base = wid * _BPW
    pltpu.sync_copy(idx_hbm.at[pl.ds(base, _BPW)], idx_v)

    def fire(j, buf):
        copies = []
        for h in range(2):
            dst = (rows_v.at[buf] if h == 0
                   else rows_sh.at[sid, buf])
            sems = vsems if h == 0 else ssems
            for g in range(_CHUNK // _L):
                vec = idx_v[pl.ds(j * 2 * _CHUNK + h * _CHUNK + g * _L, _L)]
                for k in range(_L):
                    i = g * _L + k
                    copies.append(
                        pltpu.async_copy(
                            tbl_hbm.at[pl.ds(vec[k], 1)],
                            dst.at[pl.ds(i, 1)],
                            sems[buf],
                        )
                    )
        return copies

    pending = fire(0, 0)
    out_pending = []
    for j in range(_NCHUNK):
        for o in out_pending:
            o.wait()
        nxt = []
        if j + 1 < _NCHUNK:
            nxt = fire(j + 1, (j + 1) % 2)
        for c in pending:
            c.wait()
        ob = base + j * 2 * _CHUNK
        out_pending = [
            pltpu.async_copy(
                rows_v.at[j % 2],
                out_hbm.at[pl.ds(ob, _CHUNK)], osem),
            pltpu.async_copy(
                rows_sh.at[sid, j % 2],
                out_hbm.at[pl.ds(ob + _CHUNK, _CHUNK)], osem),
        ]
        pending = nxt
    for o in out_pending:
        o.wait()


def kernel(input_words, in_embed):
    idx = input_words.astype(jnp.int32)
    return _gather_kernel(idx, in_embed)


# speedup vs baseline: 1.8350x; 1.0309x over previous
import functools

import jax
import jax.numpy as jnp
from jax import lax
from jax.experimental import pallas as pl
from jax.experimental.pallas import tpu as pltpu
from jax.experimental.pallas import tpu_sc as plsc

N_VOCAB = 1000000
N_EMBED = 64
BATCH = 16384

_info = plsc.get_sparse_core_info()
_NC, _NS, _L = _info.num_cores, _info.num_subcores, _info.num_lanes
_NW = _NC * _NS
_BPW = BATCH // _NW
_CHUNK = 64
_NCHUNK = _BPW // _CHUNK

_mesh = plsc.VectorSubcoreMesh(core_axis_name="c", subcore_axis_name="s")


@functools.partial(
    pl.kernel,
    mesh=_mesh,
    out_type=jax.ShapeDtypeStruct((BATCH, N_EMBED), jnp.float32),
    scratch_types=[
        pltpu.VMEM((_BPW,), jnp.int32),
        pltpu.VMEM((2, _CHUNK, N_EMBED), jnp.float32),
        [pltpu.SemaphoreType.DMA] * 4,
        pltpu.SemaphoreType.DMA,
    ],
)
def _gather_kernel(idx_hbm, tbl_hbm, out_hbm, idx_v, rows_v, sems, osem):
    wid = lax.axis_index("s") * _NC + lax.axis_index("c")
    base = wid * _BPW
    pltpu.sync_copy(idx_hbm.at[pl.ds(base, _BPW)], idx_v)

    def fire(j, buf):
        copies = []
        for g in range(_CHUNK // _L):
            vec = idx_v[pl.ds(j * _CHUNK + g * _L, _L)]
            for k in range(_L):
                i = g * _L + k
                copies.append(
                    pltpu.async_copy(
                        tbl_hbm.at[pl.ds(vec[k], 1)],
                        rows_v.at[buf].at[pl.ds(i, 1)],
                        sems[i % 4],
                    )
                )
        return copies

    pending = fire(0, 0)
    out_pending = None
    for j in range(_NCHUNK):
        if out_pending is not None:
            out_pending.wait()
        nxt = None
        if j + 1 < _NCHUNK:
            nxt = fire(j + 1, (j + 1) % 2)
        for c in pending:
            c.wait()
        out_pending = pltpu.async_copy(
            rows_v.at[j % 2],
            out_hbm.at[pl.ds(base + j * _CHUNK, _CHUNK)],
            osem,
        )
        pending = nxt
    out_pending.wait()


def kernel(input_words, in_embed):
    idx = input_words.astype(jnp.int32)
    return _gather_kernel(idx, in_embed)
